# Initial kernel scaffold; baseline (speedup 1.0000x reference)
import functools

import jax
import jax.numpy as jnp
from jax import lax
from jax.experimental import pallas as pl
from jax.experimental.pallas import tpu as pltpu
from jax.experimental.pallas import tpu_sc as plsc

N_NODES = 10000
N_HE = 10000
NNZ = 320000
D = 128
AUG = 144
CHUNK = 128
NUM_CHUNKS = NNZ // CHUNK
NC, NS = 2, 16
NW = NC * NS
CHUNKS_PER_TILE = -(-NUM_CHUNKS // NW)
ROWS_PER_TILE = N_NODES // NS

_MESH = plsc.VectorSubcoreMesh(core_axis_name="c", subcore_axis_name="s")


def _sc_pass_a(nidx, hidx, vals, x, zeros_e):
    @functools.partial(
        pl.kernel,
        out_type=jax.ShapeDtypeStruct((NC, N_HE, D), jnp.float32),
        mesh=_MESH,
        scratch_types=[
            pltpu.VMEM((CHUNK,), jnp.int32),
            pltpu.VMEM((CHUNK,), jnp.int32),
            pltpu.VMEM((CHUNK,), jnp.float32),
            pltpu.VMEM((CHUNK, D), jnp.float32),
            pltpu.VMEM_SHARED((N_HE, D), jnp.float32),
        ],
    )
    def k(nidx_hbm, hidx_hbm, vals_hbm, x_hbm, zeros_hbm, out_hbm,
          nidx_v, hidx_v, vals_v, rows_v, e_sh):
        cid = lax.axis_index("c")
        sid = lax.axis_index("s")
        wid = cid * NS + sid
        row0 = sid * ROWS_PER_TILE
        pltpu.sync_copy(zeros_hbm.at[pl.ds(row0, ROWS_PER_TILE)],
                        e_sh.at[pl.ds(row0, ROWS_PER_TILE)])
        plsc.subcore_barrier()

        @pl.loop(0, CHUNKS_PER_TILE)
        def _(i):
            ci = wid + i * NW

            @pl.when(ci < NUM_CHUNKS)
            def _():
                base = ci * CHUNK
                pltpu.sync_copy(nidx_hbm.at[pl.ds(base, CHUNK)], nidx_v)
                pltpu.sync_copy(hidx_hbm.at[pl.ds(base, CHUNK)], hidx_v)
                pltpu.sync_copy(vals_hbm.at[pl.ds(base, CHUNK)], vals_v)
                pltpu.sync_copy(x_hbm.at[nidx_v], rows_v)

                @pl.loop(0, CHUNK // 16)
                def _(g):
                    vv = vals_v[pl.ds(g * 16, 16)]
                    for j in range(16):
                        s = vv[j]
                        r = g * 16 + j
                        for b in range(D // 16):
                            sl = (r, pl.ds(b * 16, 16))
                            rows_v[sl] = rows_v[sl] * s

                pltpu.sync_copy(rows_v, e_sh.at[hidx_v], add=True)

        plsc.subcore_barrier()
        pltpu.sync_copy(e_sh.at[pl.ds(row0, ROWS_PER_TILE)],
                        out_hbm.at[cid].at[pl.ds(row0, ROWS_PER_TILE)])

    return k(nidx, hidx, vals, x, zeros_e)


def _sc_pass_c(nidx, hidx, f_aug, zeros_aug):
    @functools.partial(
        pl.kernel,
        out_type=jax.ShapeDtypeStruct((NC, N_NODES, AUG), jnp.float32),
        mesh=_MESH,
        scratch_types=[
            pltpu.VMEM((CHUNK,), jnp.int32),
            pltpu.VMEM((CHUNK,), jnp.int32),
            pltpu.VMEM((CHUNK, AUG), jnp.float32),
            pltpu.VMEM_SHARED((N_NODES, AUG), jnp.float32),
        ],
    )
    def k(nidx_hbm, hidx_hbm, f_hbm, zeros_hbm, out_hbm,
          nidx_v, hidx_v, rows_v, acc_sh):
        cid = lax.axis_index("c")
        sid = lax.axis_index("s")
        wid = cid * NS + sid
        row0 = sid * ROWS_PER_TILE
        pltpu.sync_copy(zeros_hbm.at[pl.ds(row0, ROWS_PER_TILE)],
                        acc_sh.at[pl.ds(row0, ROWS_PER_TILE)])
        plsc.subcore_barrier()

        @pl.loop(0, CHUNKS_PER_TILE)
        def _(i):
            ci = wid + i * NW

            @pl.when(ci < NUM_CHUNKS)
            def _():
                base = ci * CHUNK
                pltpu.sync_copy(nidx_hbm.at[pl.ds(base, CHUNK)], nidx_v)
                pltpu.sync_copy(hidx_hbm.at[pl.ds(base, CHUNK)], hidx_v)
                pltpu.sync_copy(f_hbm.at[hidx_v], rows_v)
                pltpu.sync_copy(rows_v, acc_sh.at[nidx_v], add=True)

        plsc.subcore_barrier()
        pltpu.sync_copy(acc_sh.at[pl.ds(row0, ROWS_PER_TILE)],
                        out_hbm.at[cid].at[pl.ds(row0, ROWS_PER_TILE)])

    return k(nidx, hidx, f_aug, zeros_aug)


_BLK = 400


def _tc_pass_b(e_parts, x, a1, a2):
    def body(e_ref, x_ref, a1_ref, a2_ref, f_ref, z_ref):
        e = e_ref[0] + e_ref[1]
        v = jnp.dot(e, a2_ref[...], preferred_element_type=jnp.float32)
        w = jnp.exp(v)
        wpad = jnp.concatenate(
            [w, jnp.zeros((_BLK, AUG - D - 1), jnp.float32)], axis=1)
        f_ref[...] = jnp.concatenate([e * w, wpad], axis=1)
        u = jnp.dot(x_ref[...], a1_ref[...], preferred_element_type=jnp.float32)
        z_ref[...] = 1e-16 * jnp.exp(-u)

    return pl.pallas_call(
        body,
        grid=(N_HE // _BLK,),
        in_specs=[
            pl.BlockSpec((NC, _BLK, D), lambda i: (0, i, 0)),
            pl.BlockSpec((_BLK, D), lambda i: (i, 0)),
            pl.BlockSpec((D, 1), lambda i: (0, 0)),
            pl.BlockSpec((D, 1), lambda i: (0, 0)),
        ],
        out_specs=[
            pl.BlockSpec((_BLK, AUG), lambda i: (i, 0)),
            pl.BlockSpec((_BLK, 1), lambda i: (i, 0)),
        ],
        out_shape=[
            jax.ShapeDtypeStruct((N_HE, AUG), jnp.float32),
            jax.ShapeDtypeStruct((N_NODES, 1), jnp.float32),
        ],
    )(e_parts, x, a1, a2)


def _tc_pass_d(n_parts, z):
    def body(n_ref, z_ref, o_ref):
        n = n_ref[0] + n_ref[1]
        numer = n[:, :D]
        denom = n[:, D:D + 1] + z_ref[...]
        o_ref[...] = numer / denom

    return pl.pallas_call(
        body,
        grid=(N_NODES // _BLK,),
        in_specs=[
            pl.BlockSpec((NC, _BLK, AUG), lambda i: (0, i, 0)),
            pl.BlockSpec((_BLK, 1), lambda i: (i, 0)),
        ],
        out_specs=pl.BlockSpec((_BLK, D), lambda i: (i, 0)),
        out_shape=jax.ShapeDtypeStruct((N_NODES, D), jnp.float32),
    )(n_parts, z)


def kernel(H_indices, H_values, X, a):
    nidx = H_indices[0].astype(jnp.int32)
    hidx = H_indices[1].astype(jnp.int32)
    a1 = a[:D]
    a2 = a[D:]
    zeros_e = jnp.zeros((N_HE, D), jnp.float32)
    zeros_aug = jnp.zeros((N_NODES, AUG), jnp.float32)

    e_parts = _sc_pass_a(nidx, hidx, H_values, X, zeros_e)
    f_aug, z = _tc_pass_b(e_parts, X, a1, a2)
    n_parts = _sc_pass_c(nidx, hidx, f_aug, zeros_aug)
    return _tc_pass_d(n_parts, z)


# speedup vs baseline: 6.0428x; 6.0428x over previous
import functools

import jax
import jax.numpy as jnp
from jax import lax
from jax.experimental import pallas as pl
from jax.experimental.pallas import tpu as pltpu
from jax.experimental.pallas import tpu_sc as plsc

N_NODES = 10000
N_HE = 10000
NNZ = 320000
D = 128
AUG = 144
CHUNK = 128
NUM_CHUNKS = NNZ // CHUNK
NC, NS = 2, 16
NW = NC * NS
CHUNKS_PER_TILE = -(-NUM_CHUNKS // NW)
NPAD = 10240
ROWS_PER_TILE = NPAD // NS

_MESH = plsc.VectorSubcoreMesh(core_axis_name="c", subcore_axis_name="s")


def _sc_pass_a(nidx, hidx, vals, x, zeros_e):
    @functools.partial(
        pl.kernel,
        out_type=jax.ShapeDtypeStruct((NC, NPAD, D), jnp.float32),
        mesh=_MESH,
        scratch_types=[
            pltpu.VMEM((CHUNK,), jnp.int32),
            pltpu.VMEM((CHUNK,), jnp.int32),
            pltpu.VMEM((CHUNK,), jnp.float32),
            pltpu.VMEM((CHUNK, D), jnp.float32),
            pltpu.VMEM_SHARED((NPAD, D), jnp.float32),
        ],
    )
    def k(nidx_hbm, hidx_hbm, vals_hbm, x_hbm, zeros_hbm, out_hbm,
          nidx_v, hidx_v, vals_v, rows_v, e_sh):
        cid = lax.axis_index("c")
        sid = lax.axis_index("s")
        wid = cid * NS + sid
        row0 = sid * ROWS_PER_TILE
        pltpu.sync_copy(zeros_hbm.at[pl.ds(row0, ROWS_PER_TILE)],
                        e_sh.at[pl.ds(row0, ROWS_PER_TILE)])
        plsc.subcore_barrier()

        @pl.loop(0, CHUNKS_PER_TILE)
        def _(i):
            ci = wid + i * NW

            @pl.when(ci < NUM_CHUNKS)
            def _():
                base = ci * CHUNK
                pltpu.sync_copy(nidx_hbm.at[pl.ds(base, CHUNK)], nidx_v)
                pltpu.sync_copy(hidx_hbm.at[pl.ds(base, CHUNK)], hidx_v)
                pltpu.sync_copy(vals_hbm.at[pl.ds(base, CHUNK)], vals_v)
                pltpu.sync_copy(x_hbm.at[nidx_v], rows_v)

                @pl.loop(0, CHUNK // 16)
                def _(g):
                    vv = vals_v[pl.ds(g * 16, 16)]
                    for j in range(16):
                        s = vv[j]
                        r = g * 16 + j
                        for b in range(D // 16):
                            sl = (r, pl.ds(b * 16, 16))
                            rows_v[sl] = rows_v[sl] * s

                pltpu.sync_copy(rows_v, e_sh.at[hidx_v], add=True)

        plsc.subcore_barrier()
        pltpu.sync_copy(e_sh.at[pl.ds(row0, ROWS_PER_TILE)],
                        out_hbm.at[cid].at[pl.ds(row0, ROWS_PER_TILE)])

    return k(nidx, hidx, vals, x, zeros_e)


def _sc_pass_c(nidx, hidx, f, w1d, zeros_e, zeros1):
    @functools.partial(
        pl.kernel,
        out_type=[jax.ShapeDtypeStruct((NC, NPAD, D), jnp.float32),
                  jax.ShapeDtypeStruct((NC, NPAD), jnp.float32)],
        mesh=_MESH,
        scratch_types=[
            pltpu.VMEM((CHUNK,), jnp.int32),
            pltpu.VMEM((CHUNK,), jnp.int32),
            pltpu.VMEM((CHUNK, D), jnp.float32),
            pltpu.VMEM((CHUNK,), jnp.float32),
            pltpu.VMEM_SHARED((NPAD, D), jnp.float32),
            pltpu.VMEM_SHARED((NPAD,), jnp.float32),
        ],
    )
    def k(nidx_hbm, hidx_hbm, f_hbm, w_hbm, zeros_hbm, zeros1_hbm,
          out_hbm, dout_hbm, nidx_v, hidx_v, rows_v, wg_v, acc_sh, dn_sh):
        cid = lax.axis_index("c")
        sid = lax.axis_index("s")
        wid = cid * NS + sid
        row0 = sid * ROWS_PER_TILE
        pltpu.sync_copy(zeros_hbm.at[pl.ds(row0, ROWS_PER_TILE)],
                        acc_sh.at[pl.ds(row0, ROWS_PER_TILE)])
        pltpu.sync_copy(zeros1_hbm.at[pl.ds(row0, ROWS_PER_TILE)],
                        dn_sh.at[pl.ds(row0, ROWS_PER_TILE)])
        plsc.subcore_barrier()

        @pl.loop(0, CHUNKS_PER_TILE)
        def _(i):
            ci = wid + i * NW

            @pl.when(ci < NUM_CHUNKS)
            def _():
                base = ci * CHUNK
                pltpu.sync_copy(nidx_hbm.at[pl.ds(base, CHUNK)], nidx_v)
                pltpu.sync_copy(hidx_hbm.at[pl.ds(base, CHUNK)], hidx_v)
                pltpu.sync_copy(f_hbm.at[hidx_v], rows_v)
                pltpu.sync_copy(w_hbm.at[hidx_v], wg_v)
                pltpu.sync_copy(rows_v, acc_sh.at[nidx_v], add=True)
                pltpu.sync_copy(wg_v, dn_sh.at[nidx_v], add=True)

        plsc.subcore_barrier()
        pltpu.sync_copy(acc_sh.at[pl.ds(row0, ROWS_PER_TILE)],
                        out_hbm.at[cid].at[pl.ds(row0, ROWS_PER_TILE)])
        pltpu.sync_copy(dn_sh.at[pl.ds(row0, ROWS_PER_TILE)],
                        dout_hbm.at[cid].at[pl.ds(row0, ROWS_PER_TILE)])

    return k(nidx, hidx, f, w1d, zeros_e, zeros1)


_BLK = 400


def _tc_pass_b(e_parts, x, a1, a2):
    def body(e_ref, x_ref, a1_ref, a2_ref, f_ref, w_ref, z_ref):
        e = e_ref[0] + e_ref[1]
        v = jnp.dot(e, a2_ref[...], preferred_element_type=jnp.float32)
        w = jnp.exp(v)
        f_ref[...] = e * w
        w_ref[...] = w
        u = jnp.dot(x_ref[...], a1_ref[...], preferred_element_type=jnp.float32)
        z_ref[...] = 1e-16 * jnp.exp(-u)

    return pl.pallas_call(
        body,
        grid=(N_HE // _BLK,),
        in_specs=[
            pl.BlockSpec((NC, _BLK, D), lambda i: (0, i, 0)),
            pl.BlockSpec((_BLK, D), lambda i: (i, 0)),
            pl.BlockSpec((D, 1), lambda i: (0, 0)),
            pl.BlockSpec((D, 1), lambda i: (0, 0)),
        ],
        out_specs=[
            pl.BlockSpec((_BLK, D), lambda i: (i, 0)),
            pl.BlockSpec((_BLK, 1), lambda i: (i, 0)),
            pl.BlockSpec((_BLK, 1), lambda i: (i, 0)),
        ],
        out_shape=[
            jax.ShapeDtypeStruct((N_HE, D), jnp.float32),
            jax.ShapeDtypeStruct((N_HE, 1), jnp.float32),
            jax.ShapeDtypeStruct((N_NODES, 1), jnp.float32),
        ],
    )(e_parts, x, a1, a2)


def _tc_pass_d(n_parts, d_parts, z):
    def body(n_ref, d_ref, z_ref, o_ref):
        numer = n_ref[0] + n_ref[1]
        denom = d_ref[0] + d_ref[1] + z_ref[...]
        o_ref[...] = numer / denom

    return pl.pallas_call(
        body,
        grid=(N_NODES // _BLK,),
        in_specs=[
            pl.BlockSpec((NC, _BLK, D), lambda i: (0, i, 0)),
            pl.BlockSpec((NC, _BLK, 1), lambda i: (0, i, 0)),
            pl.BlockSpec((_BLK, 1), lambda i: (i, 0)),
        ],
        out_specs=pl.BlockSpec((_BLK, D), lambda i: (i, 0)),
        out_shape=jax.ShapeDtypeStruct((N_NODES, D), jnp.float32),
    )(n_parts, d_parts, z)


def kernel(H_indices, H_values, X, a):
    nidx = H_indices[0].astype(jnp.int32)
    hidx = H_indices[1].astype(jnp.int32)
    a1 = a[:D]
    a2 = a[D:]
    zeros_e = jnp.zeros((NPAD, D), jnp.float32)
    zeros1 = jnp.zeros((NPAD,), jnp.float32)

    e_parts = _sc_pass_a(nidx, hidx, H_values, X, zeros_e)[:, :N_HE]
    f, w2, z = _tc_pass_b(e_parts, X, a1, a2)
    w1d = w2.reshape((N_HE,))
    n_parts, d_parts = _sc_pass_c(nidx, hidx, f, w1d, zeros_e, zeros1)
    n_parts = n_parts[:, :N_NODES]
    d_parts = d_parts[:, :N_NODES].reshape((NC, N_NODES, 1))
    return _tc_pass_d(n_parts, d_parts, z)
